# baseline (speedup 1.0000x reference)
import jax
import jax.numpy as jnp
from jax import lax
from jax.experimental import pallas as pl
from jax.experimental.pallas import tpu as pltpu

_F_IN, _HID, _OUT = 3, 16, 7
_LB = 256
_NQ = 4
_W1B = 0
_W2B = 64
_W3S = 128
_BCOL = 192
_B3R = 256
_REP = 264
_PROWS = 328

_IN_FP, _IN_HP = 8, 128
_IN_W1, _IN_W2, _IN_W3 = 0, _IN_FP, _IN_FP + _IN_HP
_IN_B1 = _IN_FP + 2 * _IN_HP
_IN_B2 = _IN_B1 + 8
_IN_B3 = _IN_B2 + 8


def _gcn_kernel(xs_ref, r_ref, q_ref, a_ref, w_ref, b_ref, p_ref, out_ref):
    e = r_ref.shape[2]
    f32 = jnp.float32

    xs = xs_ref[0]
    r = r_ref[0]
    q = q_ref[0]
    aa = a_ref[0]
    w = w_ref[0]
    bcol = b_ref[0]

    w1b = p_ref[_W1B:_W1B + 64, :32]
    w2b = p_ref[_W2B:_W2B + 64, :64]
    w3s = p_ref[_W3S:_W3S + 64, :]
    b1s = p_ref[_BCOL:_BCOL + 64, 0:1]
    b2s = p_ref[_BCOL:_BCOL + 64, 1:2]
    b3r = p_ref[_B3R:_B3R + 1, :]
    rep = p_ref[_REP:_REP + 64, :8]

    lane_iota = lax.broadcasted_iota(jnp.int32, (e, _LB), 1)
    ohb = (lane_iota == bcol).astype(f32)
    row_iota = lax.broadcasted_iota(jnp.int32, (_LB, e), 0)
    ohrw = jnp.where(row_iota == r, w, 0.0)

    qmf = [(q == k).astype(f32) for k in range(_NQ)]
    amf = [(aa == k).astype(f32) for k in range(_NQ)]

    iota8 = lax.broadcasted_iota(jnp.int32, (8, e), 0)
    qd = jnp.where(aa == iota8, w, 0.0)
    deg = jnp.dot(qd, ohb, preferred_element_type=f32) + 1.0
    dinv = lax.rsqrt(deg)
    dinv_s = jnp.dot(rep, dinv, preferred_element_type=f32)
    dinv2_s = dinv_s * dinv_s

    def a_hat(vt):
        vs = vt * dinv_s
        p_all = jnp.dot(vs, ohrw, preferred_element_type=f32)
        top = p_all[0:8, :] * qmf[0]
        bot = p_all[8:16, :] * qmf[0]
        for k in range(1, _NQ):
            top = top + p_all[16 * k:16 * k + 8, :] * qmf[k]
            bot = bot + p_all[16 * k + 8:16 * k + 16, :] * qmf[k]
        qmat = jnp.concatenate(
            [half * amf[k] for k in range(_NQ) for half in (top, bot)],
            axis=0)
        out_all = jnp.dot(qmat, ohb, preferred_element_type=f32)
        return out_all * dinv_s + vt * dinv2_s

    vt1 = jnp.dot(w1b, xs, preferred_element_type=f32)
    h1 = jnp.maximum(a_hat(vt1) + b1s, 0.0)
    vt2 = jnp.dot(w2b, h1, preferred_element_type=f32)
    h2 = jnp.maximum(a_hat(vt2) + b2s, 0.0)

    pooled = jnp.sum(h2, axis=1, keepdims=True)
    head = jnp.sum(pooled * w3s, axis=0, keepdims=True) + b3r
    out_ref[0] = head[:, :128]


@jax.jit
def _forward(x, edge_index, edge_weight, packed_params):
    B, N, _ = x.shape
    E = edge_index.shape[2]

    src = edge_index[:, 0, :]
    tgt = edge_index[:, 1, :]
    r_row = (src & (_LB - 1))[:, None, :]
    q_row = (src >> 8)[:, None, :]
    a_row = (tgt >> 8)[:, None, :]
    w_row = edge_weight[:, None, :]
    b_col = (tgt & (_LB - 1))[:, :, None]

    xt = jnp.zeros((B, 8, N), jnp.float32).at[:, :_F_IN, :].set(
        jnp.swapaxes(x, 1, 2))
    xs = jnp.swapaxes(xt.reshape(B, 8, _NQ, _LB), 1, 2).reshape(B, 32, _LB)

    pp = packed_params
    w1t = jnp.zeros((16, 8), jnp.float32).at[:, :_F_IN].set(
        jnp.swapaxes(pp[_IN_W1:_IN_W1 + _F_IN, :16], 0, 1))
    w2t = jnp.swapaxes(pp[_IN_W2:_IN_W2 + 16, :16], 0, 1)
    eye4 = jnp.eye(_NQ, dtype=jnp.float32)
    pbuf = jnp.zeros((_PROWS, _LB), jnp.float32)
    pbuf = pbuf.at[_W1B:_W1B + 64, :32].set(jnp.kron(eye4, w1t))
    pbuf = pbuf.at[_W2B:_W2B + 64, :64].set(jnp.kron(eye4, w2t))
    pbuf = pbuf.at[_W3S:_W3S + 64, :128].set(
        jnp.tile(pp[_IN_W3:_IN_W3 + 16, :], (_NQ, 1)))
    pbuf = pbuf.at[_BCOL:_BCOL + 64, 0].set(jnp.tile(pp[_IN_B1, :16], _NQ))
    pbuf = pbuf.at[_BCOL:_BCOL + 64, 1].set(jnp.tile(pp[_IN_B2, :16], _NQ))
    pbuf = pbuf.at[_B3R, :128].set(pp[_IN_B3, :])
    pbuf = pbuf.at[_REP:_REP + 64, :_NQ].set(
        jnp.kron(eye4, jnp.ones((16, 1))))

    out = pl.pallas_call(
        _gcn_kernel,
        out_shape=jax.ShapeDtypeStruct((B, 1, 128), jnp.float32),
        grid=(B,),
        in_specs=[
            pl.BlockSpec((1, 32, _LB), lambda g: (g, 0, 0)),
            pl.BlockSpec((1, 1, E), lambda g: (g, 0, 0)),
            pl.BlockSpec((1, 1, E), lambda g: (g, 0, 0)),
            pl.BlockSpec((1, 1, E), lambda g: (g, 0, 0)),
            pl.BlockSpec((1, 1, E), lambda g: (g, 0, 0)),
            pl.BlockSpec((1, E, 1), lambda g: (g, 0, 0)),
            pl.BlockSpec((_PROWS, _LB), lambda g: (0, 0)),
        ],
        out_specs=pl.BlockSpec((1, 1, 128), lambda g: (g, 0, 0)),
        compiler_params=pltpu.CompilerParams(
            dimension_semantics=("parallel",)),
    )(xs, r_row, q_row, a_row, w_row, b_col, pbuf)

    return out[:, 0, :_OUT]


def kernel(x, edge_index, edge_weight, packed_params):
    return _forward(x, edge_index, edge_weight, packed_params)


# speedup vs baseline: 1.3775x; 1.3775x over previous
import jax
import jax.numpy as jnp
from jax import lax
from jax.experimental import pallas as pl
from jax.experimental.pallas import tpu as pltpu

_F_IN, _HID, _OUT = 3, 16, 7
_LB = 128
_W1T = 0
_W2T = 16
_W3 = 32
_B1C = 48
_B2C = 64
_B3R = 80
_PROWS = 88

_IN_FP, _IN_HP = 8, 128
_IN_W1, _IN_W2, _IN_W3 = 0, _IN_FP, _IN_FP + _IN_HP
_IN_B1 = _IN_FP + 2 * _IN_HP
_IN_B2 = _IN_B1 + 8
_IN_B3 = _IN_B2 + 8


def _gcn_kernel(xt_ref, r_ref, q_ref, a_ref, w_ref, b_ref, p_ref, out_ref):
    n = xt_ref.shape[2]
    e = r_ref.shape[2]
    nb = n // _LB

    xt = xt_ref[0]
    r = r_ref[0]
    q = q_ref[0]
    aa = a_ref[0]
    w = w_ref[0]
    bcol = b_ref[0]

    w1t = p_ref[_W1T:_W1T + 16, :8]
    w2t = p_ref[_W2T:_W2T + 16, :16]
    w3 = p_ref[_W3:_W3 + 16, :]
    b1c = p_ref[_B1C:_B1C + 16, 0:1]
    b2c = p_ref[_B2C:_B2C + 16, 0:1]
    b3r = p_ref[_B3R:_B3R + 1, :]

    f32 = jnp.float32

    lane_iota = lax.broadcasted_iota(jnp.int32, (e, _LB), 1)
    ohb = (lane_iota == bcol).astype(f32)

    row_iota = lax.broadcasted_iota(jnp.int32, (_LB, e), 0)
    ohrw = jnp.where(row_iota == r, w, 0.0)

    amask = [(aa == k).astype(f32) for k in range(nb)]
    qmask = [(q == k).astype(f32) for k in range(nb)]

    qd = jnp.concatenate([w * amask[k] for k in range(nb)], axis=0)
    deg = jnp.dot(qd, ohb, preferred_element_type=f32) + 1.0
    dinv = lax.rsqrt(deg)
    dinv2 = dinv * dinv

    def a_hat(vt):
        vs = jnp.concatenate(
            [vt[:, k * _LB:(k + 1) * _LB] * dinv[k:k + 1, :]
             for k in range(nb)], axis=1)
        wmat = jnp.concatenate(
            [vs[:, k * _LB:(k + 1) * _LB] for k in range(nb)], axis=0)
        p_all = jnp.dot(wmat, ohrw, preferred_element_type=f32)
        msg = p_all[0:16, :] * qmask[0]
        for k in range(1, nb):
            msg = msg + p_all[16 * k:16 * (k + 1), :] * qmask[k]
        qmat = jnp.concatenate(
            [msg * amask[k] for k in range(nb)], axis=0)
        out_all = jnp.dot(qmat, ohb, preferred_element_type=f32)
        return jnp.concatenate(
            [out_all[16 * k:16 * (k + 1), :] * dinv[k:k + 1, :]
             + vt[:, k * _LB:(k + 1) * _LB] * dinv2[k:k + 1, :]
             for k in range(nb)], axis=1)

    vt1 = jnp.dot(w1t, xt, preferred_element_type=f32)
    h1 = jnp.maximum(a_hat(vt1) + b1c, 0.0)
    vt2 = jnp.dot(w2t, h1, preferred_element_type=f32)
    h2 = jnp.maximum(a_hat(vt2) + b2c, 0.0)

    pooled = jnp.sum(h2, axis=1, keepdims=True)
    out_ref[0] = jnp.sum(pooled * w3, axis=0, keepdims=True) + b3r


@jax.jit
def _forward(x, edge_index, edge_weight, packed_params):
    B, N, _ = x.shape
    E = edge_index.shape[2]

    src = edge_index[:, 0, :]
    tgt = edge_index[:, 1, :]
    r_row = (src & (_LB - 1))[:, None, :]
    q_row = (src >> 7)[:, None, :]
    a_row = (tgt >> 7)[:, None, :]
    b_col = (tgt & (_LB - 1))[:, :, None]
    w_row = edge_weight[:, None, :]

    xt = jnp.zeros((B, 8, N), jnp.float32).at[:, :_F_IN, :].set(
        jnp.swapaxes(x, 1, 2))

    pp = packed_params
    pbuf = jnp.zeros((_PROWS, 128), jnp.float32)
    pbuf = pbuf.at[_W1T:_W1T + 16, :_F_IN].set(
        jnp.swapaxes(pp[_IN_W1:_IN_W1 + _F_IN, :16], 0, 1))
    pbuf = pbuf.at[_W2T:_W2T + 16, :16].set(
        jnp.swapaxes(pp[_IN_W2:_IN_W2 + 16, :16], 0, 1))
    pbuf = pbuf.at[_W3:_W3 + 16, :].set(pp[_IN_W3:_IN_W3 + 16, :])
    pbuf = pbuf.at[_B1C:_B1C + 16, 0].set(pp[_IN_B1, :16])
    pbuf = pbuf.at[_B2C:_B2C + 16, 0].set(pp[_IN_B2, :16])
    pbuf = pbuf.at[_B3R, :].set(pp[_IN_B3, :])

    out = pl.pallas_call(
        _gcn_kernel,
        out_shape=jax.ShapeDtypeStruct((B, 1, 128), jnp.float32),
        grid=(B,),
        in_specs=[
            pl.BlockSpec((1, 8, N), lambda g: (g, 0, 0)),
            pl.BlockSpec((1, 1, E), lambda g: (g, 0, 0)),
            pl.BlockSpec((1, 1, E), lambda g: (g, 0, 0)),
            pl.BlockSpec((1, 1, E), lambda g: (g, 0, 0)),
            pl.BlockSpec((1, 1, E), lambda g: (g, 0, 0)),
            pl.BlockSpec((1, E, 1), lambda g: (g, 0, 0)),
            pl.BlockSpec((_PROWS, 128), lambda g: (0, 0)),
        ],
        out_specs=pl.BlockSpec((1, 1, 128), lambda g: (g, 0, 0)),
        compiler_params=pltpu.CompilerParams(
            dimension_semantics=("parallel",)),
    )(xt, r_row, q_row, a_row, w_row, b_col, pbuf)

    return out[:, 0, :_OUT]


def kernel(x, edge_index, edge_weight, packed_params):
    return _forward(x, edge_index, edge_weight, packed_params)


# speedup vs baseline: 1.9896x; 1.4443x over previous
import jax
import jax.numpy as jnp
from jax import lax
from jax.experimental import pallas as pl
from jax.experimental.pallas import tpu as pltpu

_F_IN, _HID, _OUT = 3, 16, 7
_LB = 128
_W1T = 0
_W2T = 16
_W3 = 32
_B1C = 48
_B2C = 64
_B3R = 80
_PROWS = 88

_IN_FP, _IN_HP = 8, 128
_IN_W1, _IN_W2, _IN_W3 = 0, _IN_FP, _IN_FP + _IN_HP
_IN_B1 = _IN_FP + 2 * _IN_HP
_IN_B2 = _IN_B1 + 8
_IN_B3 = _IN_B2 + 8


def _gcn_kernel(xp_ref, r_ref, q_ref, a_ref, w_ref, b_ref, p_ref, out_ref):
    n = xp_ref.shape[1]
    e = r_ref.shape[2]
    nb = n // _LB

    xp = xp_ref[0]
    r = r_ref[0]
    q = q_ref[0]
    aa = a_ref[0]
    w = w_ref[0]
    b = b_ref[0]

    w1t = p_ref[_W1T:_W1T + 16, :8]
    w2t = p_ref[_W2T:_W2T + 16, :16]
    w3 = p_ref[_W3:_W3 + 16, :]
    b1c = p_ref[_B1C:_B1C + 16, 0:1]
    b2c = p_ref[_B2C:_B2C + 16, 0:1]
    b3r = p_ref[_B3R:_B3R + 1, :]

    f32 = jnp.float32
    _t = (((1,), (1,)), ((), ()))

    row_iota = lax.broadcasted_iota(jnp.int32, (_LB, e), 0)
    ohbt = (row_iota == b).astype(f32)
    ohrw = jnp.where(row_iota == r, w, 0.0)

    amask = [(aa == k).astype(f32) for k in range(nb)]
    qmask = [(q == k).astype(f32) for k in range(nb)]

    qd = jnp.concatenate([w * amask[k] for k in range(nb)], axis=0)
    deg = lax.dot_general(qd, ohbt, _t,
                          preferred_element_type=f32) + 1.0
    dinv = lax.rsqrt(deg)
    dinv2 = dinv * dinv

    def a_hat(vt):
        vs = jnp.concatenate(
            [vt[:, k * _LB:(k + 1) * _LB] * dinv[k:k + 1, :]
             for k in range(nb)], axis=1)
        wmat = jnp.concatenate(
            [vs[:, k * _LB:(k + 1) * _LB] for k in range(nb)], axis=0)
        p_all = jnp.dot(wmat, ohrw, preferred_element_type=f32)
        msg = p_all[0:16, :] * qmask[0]
        for k in range(1, nb):
            msg = msg + p_all[16 * k:16 * (k + 1), :] * qmask[k]
        qmat = jnp.concatenate(
            [msg * amask[k] for k in range(nb)], axis=0)
        out_all = lax.dot_general(qmat, ohbt, _t,
                                  preferred_element_type=f32)
        return jnp.concatenate(
            [out_all[16 * k:16 * (k + 1), :] * dinv[k:k + 1, :]
             + vt[:, k * _LB:(k + 1) * _LB] * dinv2[k:k + 1, :]
             for k in range(nb)], axis=1)

    vt1 = lax.dot_general(w1t, xp, _t, preferred_element_type=f32)
    h1 = jnp.maximum(a_hat(vt1) + b1c, 0.0)
    vt2 = jnp.dot(w2t, h1, preferred_element_type=f32)
    h2 = jnp.maximum(a_hat(vt2) + b2c, 0.0)

    pooled = jnp.sum(h2, axis=1, keepdims=True)
    out_ref[0] = jnp.sum(pooled * w3, axis=0, keepdims=True) + b3r


@jax.jit
def _forward(x, edge_index, edge_weight, packed_params):
    B, N, _ = x.shape
    E = edge_index.shape[2]

    src = edge_index[:, 0, :]
    tgt = edge_index[:, 1, :]
    r_row = (src & (_LB - 1))[:, None, :]
    q_row = (src >> 7)[:, None, :]
    a_row = (tgt >> 7)[:, None, :]
    b_row = (tgt & (_LB - 1))[:, None, :]
    w_row = edge_weight[:, None, :]

    xpad = jnp.zeros((B, N, 8), jnp.float32).at[:, :, :_F_IN].set(x)

    pp = packed_params
    pbuf = jnp.zeros((_PROWS, 128), jnp.float32)
    pbuf = pbuf.at[_W1T:_W1T + 16, :_F_IN].set(
        jnp.swapaxes(pp[_IN_W1:_IN_W1 + _F_IN, :16], 0, 1))
    pbuf = pbuf.at[_W2T:_W2T + 16, :16].set(
        jnp.swapaxes(pp[_IN_W2:_IN_W2 + 16, :16], 0, 1))
    pbuf = pbuf.at[_W3:_W3 + 16, :].set(pp[_IN_W3:_IN_W3 + 16, :])
    pbuf = pbuf.at[_B1C:_B1C + 16, 0].set(pp[_IN_B1, :16])
    pbuf = pbuf.at[_B2C:_B2C + 16, 0].set(pp[_IN_B2, :16])
    pbuf = pbuf.at[_B3R, :].set(pp[_IN_B3, :])

    out = pl.pallas_call(
        _gcn_kernel,
        out_shape=jax.ShapeDtypeStruct((B, 1, 128), jnp.float32),
        grid=(B,),
        in_specs=[
            pl.BlockSpec((1, N, 8), lambda g: (g, 0, 0)),
            pl.BlockSpec((1, 1, E), lambda g: (g, 0, 0)),
            pl.BlockSpec((1, 1, E), lambda g: (g, 0, 0)),
            pl.BlockSpec((1, 1, E), lambda g: (g, 0, 0)),
            pl.BlockSpec((1, 1, E), lambda g: (g, 0, 0)),
            pl.BlockSpec((1, 1, E), lambda g: (g, 0, 0)),
            pl.BlockSpec((_PROWS, 128), lambda g: (0, 0)),
        ],
        out_specs=pl.BlockSpec((1, 1, 128), lambda g: (g, 0, 0)),
        compiler_params=pltpu.CompilerParams(
            dimension_semantics=("parallel",)),
    )(xpad, r_row, q_row, a_row, w_row, b_row, pbuf)

    return out[:, 0, :_OUT]


def kernel(x, edge_index, edge_weight, packed_params):
    return _forward(x, edge_index, edge_weight, packed_params)


# speedup vs baseline: 1.9941x; 1.0023x over previous
import jax
import jax.numpy as jnp
from jax import lax
from jax.experimental import pallas as pl
from jax.experimental.pallas import tpu as pltpu

_F_IN, _HID, _OUT = 3, 16, 7
_LB = 128
_W1T = 0
_W2T = 16
_W3 = 32
_B1C = 48
_B2C = 64
_B3R = 80
_PROWS = 88

_IN_FP, _IN_HP = 8, 128
_IN_W1, _IN_W2, _IN_W3 = 0, _IN_FP, _IN_FP + _IN_HP
_IN_B1 = _IN_FP + 2 * _IN_HP
_IN_B2 = _IN_B1 + 8
_IN_B3 = _IN_B2 + 8


def _gcn_kernel(xp_ref, r_ref, q_ref, a_ref, w_ref, b_ref, p_ref, out_ref):
    n = xp_ref.shape[1]
    e = r_ref.shape[2]
    nb = n // _LB

    xp = xp_ref[0]
    r = r_ref[0]
    q = q_ref[0]
    aa = a_ref[0]
    w = w_ref[0]
    b = b_ref[0]

    w1t = p_ref[_W1T:_W1T + 16, :8]
    w2t = p_ref[_W2T:_W2T + 16, :16]
    w3 = p_ref[_W3:_W3 + 16, :]
    b1c = p_ref[_B1C:_B1C + 16, 0:1]
    b2c = p_ref[_B2C:_B2C + 16, 0:1]
    b3r = p_ref[_B3R:_B3R + 1, :]

    f32 = jnp.float32
    _t = (((1,), (1,)), ((), ()))

    row_iota = lax.broadcasted_iota(jnp.int32, (_LB, e), 0)
    ohbt = (row_iota == b).astype(f32)
    ohrw = jnp.where(row_iota == r, w, 0.0)

    amask = [(aa == k).astype(f32) for k in range(nb)]
    qmask = [(q == k).astype(f32) for k in range(nb)]

    iota8 = lax.broadcasted_iota(jnp.int32, (8, e), 0)
    qd = jnp.where(aa == iota8, w, 0.0)
    deg = lax.dot_general(qd, ohbt, _t,
                          preferred_element_type=f32) + 1.0
    dinv = lax.rsqrt(deg)
    dinv2 = dinv * dinv

    def a_hat(vt):
        vs = jnp.concatenate(
            [vt[:, k * _LB:(k + 1) * _LB] * dinv[k:k + 1, :]
             for k in range(nb)], axis=1)
        wmat = jnp.concatenate(
            [vs[:, k * _LB:(k + 1) * _LB] for k in range(nb)], axis=0)
        p_all = jnp.dot(wmat, ohrw, preferred_element_type=f32)
        msg = p_all[0:16, :] * qmask[0]
        for k in range(1, nb):
            msg = msg + p_all[16 * k:16 * (k + 1), :] * qmask[k]
        qmat = jnp.concatenate(
            [msg * amask[k] for k in range(nb)], axis=0)
        out_all = lax.dot_general(qmat, ohbt, _t,
                                  preferred_element_type=f32)
        return jnp.concatenate(
            [out_all[16 * k:16 * (k + 1), :] * dinv[k:k + 1, :]
             + vt[:, k * _LB:(k + 1) * _LB] * dinv2[k:k + 1, :]
             for k in range(nb)], axis=1)

    vt1 = lax.dot_general(w1t, xp, _t, preferred_element_type=f32)
    h1 = jnp.maximum(a_hat(vt1) + b1c, 0.0)
    vt2 = jnp.dot(w2t, h1, preferred_element_type=f32)
    h2 = jnp.maximum(a_hat(vt2) + b2c, 0.0)

    pooled = jnp.sum(h2, axis=1, keepdims=True)
    out_ref[0] = jnp.sum(pooled * w3, axis=0, keepdims=True) + b3r


@jax.jit
def _forward(x, edge_index, edge_weight, packed_params):
    B, N, _ = x.shape
    E = edge_index.shape[2]

    src = edge_index[:, 0, :]
    tgt = edge_index[:, 1, :]
    r_row = (src & (_LB - 1))[:, None, :]
    q_row = (src >> 7)[:, None, :]
    a_row = (tgt >> 7)[:, None, :]
    b_row = (tgt & (_LB - 1))[:, None, :]
    w_row = edge_weight[:, None, :]

    xpad = jnp.zeros((B, N, 8), jnp.float32).at[:, :, :_F_IN].set(x)

    pp = packed_params
    pbuf = jnp.zeros((_PROWS, 128), jnp.float32)
    pbuf = pbuf.at[_W1T:_W1T + 16, :_F_IN].set(
        jnp.swapaxes(pp[_IN_W1:_IN_W1 + _F_IN, :16], 0, 1))
    pbuf = pbuf.at[_W2T:_W2T + 16, :16].set(
        jnp.swapaxes(pp[_IN_W2:_IN_W2 + 16, :16], 0, 1))
    pbuf = pbuf.at[_W3:_W3 + 16, :].set(pp[_IN_W3:_IN_W3 + 16, :])
    pbuf = pbuf.at[_B1C:_B1C + 16, 0].set(pp[_IN_B1, :16])
    pbuf = pbuf.at[_B2C:_B2C + 16, 0].set(pp[_IN_B2, :16])
    pbuf = pbuf.at[_B3R, :].set(pp[_IN_B3, :])

    out = pl.pallas_call(
        _gcn_kernel,
        out_shape=jax.ShapeDtypeStruct((B, 1, 128), jnp.float32),
        grid=(B,),
        in_specs=[
            pl.BlockSpec((1, N, 8), lambda g: (g, 0, 0)),
            pl.BlockSpec((1, 1, E), lambda g: (g, 0, 0)),
            pl.BlockSpec((1, 1, E), lambda g: (g, 0, 0)),
            pl.BlockSpec((1, 1, E), lambda g: (g, 0, 0)),
            pl.BlockSpec((1, 1, E), lambda g: (g, 0, 0)),
            pl.BlockSpec((1, 1, E), lambda g: (g, 0, 0)),
            pl.BlockSpec((_PROWS, 128), lambda g: (0, 0)),
        ],
        out_specs=pl.BlockSpec((1, 1, 128), lambda g: (g, 0, 0)),
        compiler_params=pltpu.CompilerParams(
            dimension_semantics=("parallel",)),
    )(xpad, r_row, q_row, a_row, w_row, b_row, pbuf)

    return out[:, 0, :_OUT]


def kernel(x, edge_index, edge_weight, packed_params):
    return _forward(x, edge_index, edge_weight, packed_params)
